# baseline (speedup 1.0000x reference)
import jax
import jax.numpy as jnp
from jax.experimental import pallas as pl
from jax.experimental.pallas import tpu as pltpu

N_TOK = 32 * 32 * 32
B = 32
HW = 32 * 32
D = 64
K = 1024
T = 2048
GRID = N_TOK // T
COMMIT = 0.25


def _vq_block(x_ref, emb_ref, embt2_ref, colf_ref,
              qout_ref, enc_ref, idx_ref, loss_ref, ppl_ref,
              hist_acc, loss_acc):
    step = pl.program_id(0)

    @pl.when(step == 0)
    def _init():
        hist_acc[:] = jnp.zeros_like(hist_acc)
        loss_acc[0, 0] = 0.0

    x = x_ref[:]
    embt2 = embt2_ref[:]
    esq = 0.25 * jnp.sum(embt2 * embt2, axis=0, keepdims=True)
    xsq = jnp.sum(x * x, axis=1, keepdims=True)
    dots2 = jnp.dot(x, embt2, preferred_element_type=jnp.float32)
    dist = xsq + esq + dots2

    dmin = jnp.min(dist, axis=1, keepdims=True)
    colf = colf_ref[:]
    idxf = jnp.min(jnp.where(dist == dmin, colf, float(K)),
                   axis=1, keepdims=True)
    idx_ref[:] = idxf.astype(jnp.int32)

    onehot = (colf == idxf).astype(jnp.float32)
    enc_ref[:] = onehot
    q = jnp.dot(onehot, emb_ref[:], preferred_element_type=jnp.float32)
    q_cm = q.T
    qout_ref[0] = q_cm[:, :HW]
    qout_ref[1] = q_cm[:, HW:]

    diff = q - x
    loss_acc[0, 0] += jnp.sum(diff * diff)
    ones8 = jnp.ones((8, T), jnp.float32)
    hist_acc[:] += jnp.dot(ones8, onehot,
                           preferred_element_type=jnp.float32)[:1]

    @pl.when(step == GRID - 1)
    def _fin():
        loss_ref[:] = jnp.full(
            (1, 1), loss_acc[0, 0] * ((1.0 + COMMIT) / (N_TOK * D)),
            jnp.float32)
        p = hist_acc[:] * (1.0 / N_TOK)
        ent = jnp.sum(p * jnp.log(p + 1e-10))
        ppl_ref[:] = jnp.full((1, 1), jnp.exp(-ent), jnp.float32)


def kernel(inputs, embedding):
    x = jnp.transpose(inputs, (0, 2, 3, 1)).reshape(N_TOK, D)
    embt2 = -2.0 * embedding.T
    colf = jnp.arange(K, dtype=jnp.float32)[None, :]

    qflat, enc, idx2, loss2, ppl2 = pl.pallas_call(
        _vq_block,
        grid=(GRID,),
        in_specs=[
            pl.BlockSpec((T, D), lambda i: (i, 0)),
            pl.BlockSpec((K, D), lambda i: (0, 0)),
            pl.BlockSpec((D, K), lambda i: (0, 0)),
            pl.BlockSpec((1, K), lambda i: (0, 0)),
        ],
        out_specs=[
            pl.BlockSpec((2, D, HW), lambda i: (i, 0, 0)),
            pl.BlockSpec((T, K), lambda i: (i, 0)),
            pl.BlockSpec((T, 1), lambda i: (i, 0)),
            pl.BlockSpec((1, 1), lambda i: (0, 0)),
            pl.BlockSpec((1, 1), lambda i: (0, 0)),
        ],
        out_shape=[
            jax.ShapeDtypeStruct((B, D, HW), jnp.float32),
            jax.ShapeDtypeStruct((N_TOK, K), jnp.float32),
            jax.ShapeDtypeStruct((N_TOK, 1), jnp.int32),
            jax.ShapeDtypeStruct((1, 1), jnp.float32),
            jax.ShapeDtypeStruct((1, 1), jnp.float32),
        ],
        scratch_shapes=[
            pltpu.VMEM((1, K), jnp.float32),
            pltpu.SMEM((1, 1), jnp.float32),
        ],
    )(x, embedding, embt2, colf)

    quantized_out = qflat.reshape(B, D, 32, 32)
    return (loss2[0, 0], quantized_out, ppl2[0, 0],
            enc, idx2.reshape(N_TOK))


# speedup vs baseline: 1.1407x; 1.1407x over previous
import jax
import jax.numpy as jnp
from jax.experimental import pallas as pl
from jax.experimental.pallas import tpu as pltpu

N_TOK = 32 * 32 * 32
D = 64
K = 1024
T = 4096
GRID = N_TOK // T
COMMIT = 0.25


def _vq_block(x_ref, emb_ref, embt2_ref, colf_ref,
              qout_ref, enc_ref, idx_ref, loss_ref, ppl_ref,
              hist_acc, loss_acc):
    step = pl.program_id(0)

    @pl.when(step == 0)
    def _init():
        hist_acc[:] = jnp.zeros_like(hist_acc)
        loss_acc[0, 0] = 0.0

    x = x_ref[:]
    embt2 = embt2_ref[:]
    esq = 0.25 * jnp.sum(embt2 * embt2, axis=0, keepdims=True)
    xsq = jnp.sum(x * x, axis=1, keepdims=True)
    dots2 = jnp.dot(x, embt2, preferred_element_type=jnp.float32)
    dist = xsq + esq + dots2

    dmin = jnp.min(dist, axis=1, keepdims=True)
    colf = colf_ref[:]
    idxf = jnp.min(jnp.where(dist == dmin, colf, float(K)),
                   axis=1, keepdims=True)
    idx_ref[:] = idxf.astype(jnp.int32)

    onehot = (colf == idxf).astype(jnp.float32)
    enc_ref[:] = onehot
    q = jnp.dot(onehot, emb_ref[:], preferred_element_type=jnp.float32)
    qout_ref[:] = q

    diff = q - x
    loss_acc[0, 0] += jnp.sum(diff * diff)
    ones8 = jnp.ones((8, T), jnp.float32)
    hist_acc[:] += jnp.dot(ones8, onehot,
                           preferred_element_type=jnp.float32)[:1]

    @pl.when(step == GRID - 1)
    def _fin():
        loss_ref[:] = jnp.full(
            (1, 1), loss_acc[0, 0] * ((1.0 + COMMIT) / (N_TOK * D)),
            jnp.float32)
        p = hist_acc[:] * (1.0 / N_TOK)
        ent = jnp.sum(p * jnp.log(p + 1e-10))
        ppl_ref[:] = jnp.full((1, 1), jnp.exp(-ent), jnp.float32)


def kernel(inputs, embedding):
    x = jnp.transpose(inputs, (0, 2, 3, 1)).reshape(N_TOK, D)
    embt2 = -2.0 * embedding.T
    colf = jnp.arange(K, dtype=jnp.float32)[None, :]

    qflat, enc, idx2, loss2, ppl2 = pl.pallas_call(
        _vq_block,
        grid=(GRID,),
        in_specs=[
            pl.BlockSpec((T, D), lambda i: (i, 0)),
            pl.BlockSpec((K, D), lambda i: (0, 0)),
            pl.BlockSpec((D, K), lambda i: (0, 0)),
            pl.BlockSpec((1, K), lambda i: (0, 0)),
        ],
        out_specs=[
            pl.BlockSpec((T, D), lambda i: (i, 0)),
            pl.BlockSpec((T, K), lambda i: (i, 0)),
            pl.BlockSpec((T, 1), lambda i: (i, 0)),
            pl.BlockSpec((1, 1), lambda i: (0, 0)),
            pl.BlockSpec((1, 1), lambda i: (0, 0)),
        ],
        out_shape=[
            jax.ShapeDtypeStruct((N_TOK, D), jnp.float32),
            jax.ShapeDtypeStruct((N_TOK, K), jnp.float32),
            jax.ShapeDtypeStruct((N_TOK, 1), jnp.int32),
            jax.ShapeDtypeStruct((1, 1), jnp.float32),
            jax.ShapeDtypeStruct((1, 1), jnp.float32),
        ],
        scratch_shapes=[
            pltpu.VMEM((1, K), jnp.float32),
            pltpu.SMEM((1, 1), jnp.float32),
        ],
        compiler_params=pltpu.CompilerParams(
            vmem_limit_bytes=100 * 1024 * 1024),
    )(x, embedding, embt2, colf)

    quantized_out = jnp.transpose(
        qflat.reshape(32, 32, 32, D), (0, 3, 1, 2))
    return (loss2[0, 0], quantized_out, ppl2[0, 0],
            enc, idx2.reshape(N_TOK))
